# Initial kernel scaffold; baseline (speedup 1.0000x reference)
import jax
import jax.numpy as jnp
from jax.experimental import pallas as pl


def kernel(indices, table):
    raise NotImplementedError("write your pallas kernel here")


# speedup vs baseline: 1.9537x; 1.9537x over previous
import functools

import jax
import jax.numpy as jnp
from jax import lax
from jax.experimental import pallas as pl
from jax.experimental.pallas import tpu as pltpu
from jax.experimental.pallas import tpu_sc as plsc

V = 8192
D = 8192
NW = 32
NPW = 256
C = 2
NCH = NPW // C
NBUF = 4
K = 2

_MESH = plsc.VectorSubcoreMesh(core_axis_name="c", subcore_axis_name="s")


@functools.partial(
    pl.kernel,
    out_type=jax.ShapeDtypeStruct((NW * NPW, D), jnp.float32),
    mesh=_MESH,
    scratch_types=[
        pltpu.VMEM((NCH, C), jnp.int32),
        [pltpu.VMEM((C, D), jnp.float32) for _ in range(NBUF)],
        [pltpu.SemaphoreType.DMA for _ in range(NBUF)],
        [pltpu.SemaphoreType.DMA for _ in range(NBUF)],
    ],
)
def _sc_gather(idx_hbm, table_hbm, out_hbm, idx_v, bufs, sem_in, sem_out):
    wid = lax.axis_index("s") * 2 + lax.axis_index("c")
    base = wid * NPW

    def gather(g, b):
        return pltpu.make_async_copy(
            table_hbm.at[idx_v.at[g]], bufs[b], sem_in[b])

    def scatter(g, b):
        return pltpu.make_async_copy(
            bufs[b], out_hbm.at[pl.ds(base + g * C, C)], sem_out[b])

    pltpu.sync_copy(idx_hbm.at[wid], idx_v)

    for g in range(K):
        gather(g, g).start()

    @pl.loop(0, NCH // NBUF)
    def _(go):
        for b in range(NBUF):
            g = go * NBUF + b
            gp = g + K
            bp = (b + K) % NBUF

            @pl.when(gp < NCH)
            def _():
                @pl.when(gp >= NBUF)
                def _():
                    scatter(gp - NBUF, bp).wait()

                gather(gp, bp).start()

            gather(g, b).wait()
            scatter(g, b).start()

    for b in range(NBUF):
        scatter(NCH - NBUF + b, b).wait()


def kernel(indices, table):
    flat = indices.reshape(-1).astype(jnp.int32).reshape(NW, NCH, C)
    out = _sc_gather(flat, table)
    return out.reshape(indices.shape + (D,))


# speedup vs baseline: 1.9653x; 1.0059x over previous
import functools

import jax
import jax.numpy as jnp
from jax import lax
from jax.experimental import pallas as pl
from jax.experimental.pallas import tpu as pltpu
from jax.experimental.pallas import tpu_sc as plsc

V = 8192
D = 8192
NW = 32
NPW = 256
C = 4
NCH = NPW // C
NBUF = 3
K = 1
NGRP = -(-NCH // NBUF)

_MESH = plsc.VectorSubcoreMesh(core_axis_name="c", subcore_axis_name="s")


@functools.partial(
    pl.kernel,
    out_type=jax.ShapeDtypeStruct((NW * NPW, D), jnp.float32),
    mesh=_MESH,
    scratch_types=[
        pltpu.VMEM((NCH, C), jnp.int32),
        [pltpu.VMEM((C, D), jnp.float32) for _ in range(NBUF)],
        [pltpu.SemaphoreType.DMA for _ in range(NBUF)],
        [pltpu.SemaphoreType.DMA for _ in range(NBUF)],
    ],
)
def _sc_gather(idx_hbm, table_hbm, out_hbm, idx_v, bufs, sem_in, sem_out):
    wid = lax.axis_index("s") * 2 + lax.axis_index("c")
    base = wid * NPW

    def gather(g, b):
        return pltpu.make_async_copy(
            table_hbm.at[idx_v.at[g]], bufs[b], sem_in[b])

    def scatter(g, b):
        return pltpu.make_async_copy(
            bufs[b], out_hbm.at[pl.ds(base + g * C, C)], sem_out[b])

    pltpu.sync_copy(idx_hbm.at[wid], idx_v)

    for g in range(K):
        gather(g, g).start()

    @pl.loop(0, NGRP)
    def _(go):
        for b in range(NBUF):
            g = go * NBUF + b

            @pl.when(g < NCH)
            def _():
                gp = g + K
                bp = (b + K) % NBUF

                @pl.when(gp < NCH)
                def _():
                    @pl.when(gp >= NBUF)
                    def _():
                        scatter(gp - NBUF, bp).wait()

                    gather(gp, bp).start()

                gather(g, b).wait()
                scatter(g, b).start()

    for j in range(NBUF):
        g_last = NCH - NBUF + j
        scatter(g_last, g_last % NBUF).wait()


def kernel(indices, table):
    flat = indices.reshape(-1).astype(jnp.int32).reshape(NW, NCH, C)
    out = _sc_gather(flat, table)
    return out.reshape(indices.shape + (D,))
